# baseline (device time: 123834 ns/iter reference)
import jax
import jax.numpy as jnp
from jax import lax
from jax.experimental import pallas as pl
from jax.experimental.pallas import tpu as pltpu

N_DEV = 16


def kernel(x, w_mat):
    m, k_per = x.shape
    _, n = w_mat.shape
    m_per = m // N_DEV

    def body(x_ref, w_ref, out_ref, send_buf, recv_buf, send_sems, recv_sems):
        my = lax.axis_index("i")
        left = lax.rem(my + (N_DEV - 1), N_DEV)
        right = lax.rem(my + 1, N_DEV)

        barrier = pltpu.get_barrier_semaphore()
        for nbr in (left, right):
            pl.semaphore_signal(
                barrier, inc=1,
                device_id=(nbr,), device_id_type=pl.DeviceIdType.MESH,
            )
        pl.semaphore_wait(barrier, 2)

        w = w_ref[...].astype(jnp.bfloat16)

        def partial_chunk(c):
            xc = x_ref[pl.ds(c * m_per, m_per), :].astype(jnp.bfloat16)
            return jnp.dot(xc, w, preferred_element_type=jnp.float32)

        for s in range(N_DEV - 1):
            c = lax.rem(my + (N_DEV - 1 - s), N_DEV)
            acc = partial_chunk(c)
            if s > 0:
                acc = acc + recv_buf[s - 1].astype(jnp.float32)
            send_buf[...] = acc.astype(jnp.bfloat16)
            rdma = pltpu.make_async_remote_copy(
                src_ref=send_buf,
                dst_ref=recv_buf.at[s],
                send_sem=send_sems.at[s],
                recv_sem=recv_sems.at[s],
                device_id=(right,),
                device_id_type=pl.DeviceIdType.MESH,
            )
            rdma.start()
            rdma.wait()

        final = partial_chunk(my) + recv_buf[N_DEV - 2].astype(jnp.float32)
        out_ref[...] = final * jax.nn.sigmoid(final)

    return pl.pallas_call(
        body,
        out_shape=jax.ShapeDtypeStruct((m_per, n), jnp.float32),
        in_specs=[
            pl.BlockSpec(memory_space=pltpu.VMEM),
            pl.BlockSpec(memory_space=pltpu.VMEM),
        ],
        out_specs=pl.BlockSpec(memory_space=pltpu.VMEM),
        scratch_shapes=[
            pltpu.VMEM((m_per, n), jnp.bfloat16),
            pltpu.VMEM((N_DEV - 1, m_per, n), jnp.bfloat16),
            pltpu.SemaphoreType.DMA((N_DEV - 1,)),
            pltpu.SemaphoreType.DMA((N_DEV - 1,)),
        ],
        compiler_params=pltpu.CompilerParams(collective_id=0),
    )(x, w_mat)


# device time: 82564 ns/iter; 1.4999x vs baseline; 1.4999x over previous
import jax
import jax.numpy as jnp
from jax import lax
from jax.experimental import pallas as pl
from jax.experimental.pallas import tpu as pltpu

N_DEV = 16

PERM = [0, 1, 5, 9, 13, 14, 10, 6, 2, 3, 7, 11, 15, 12, 8, 4]
INV = [PERM.index(l) for l in range(N_DEV)]
NXT_CW = [PERM[(INV[l] + 1) % N_DEV] for l in range(N_DEV)]
NXT_CCW = [PERM[(INV[l] - 1) % N_DEV] for l in range(N_DEV)]


def _lut(table, idx):
    acc = jnp.int32(0)
    for j, v in enumerate(table):
        acc = acc + jnp.where(idx == j, jnp.int32(v), jnp.int32(0))
    return acc


def kernel(x, w_mat):
    m, k_per = x.shape
    _, n = w_mat.shape
    m_per = m // N_DEV
    nh = n // 2

    def body(x_ref, w_ref, out_ref,
             send_cw, send_ccw, recv_cw, recv_ccw,
             send_sems, recv_sems):
        my = lax.axis_index("i")
        r = _lut(INV, my)
        nxt_cw = _lut(NXT_CW, my)
        nxt_ccw = _lut(NXT_CCW, my)

        barrier = pltpu.get_barrier_semaphore()
        for nbr in (nxt_cw, nxt_ccw):
            pl.semaphore_signal(
                barrier, inc=1,
                device_id=(nbr,), device_id_type=pl.DeviceIdType.MESH,
            )
        pl.semaphore_wait(barrier, 2)

        w = w_ref[...].astype(jnp.bfloat16)

        def partial_chunk(ring_chunk, col0):
            row = _lut(PERM, ring_chunk)
            xc = x_ref[pl.ds(row * m_per, m_per), :].astype(jnp.bfloat16)
            return jnp.dot(xc, w[:, col0:col0 + nh],
                           preferred_element_type=jnp.float32)

        for s in range(N_DEV - 1):
            c_cw = lax.rem(r + (2 * N_DEV - 1 - s), N_DEV)
            acc = partial_chunk(c_cw, 0)
            if s > 0:
                acc = acc + recv_cw[s - 1].astype(jnp.float32)
            send_cw[...] = acc.astype(jnp.bfloat16)
            rdma_cw = pltpu.make_async_remote_copy(
                src_ref=send_cw,
                dst_ref=recv_cw.at[s],
                send_sem=send_sems.at[s, 0],
                recv_sem=recv_sems.at[s, 0],
                device_id=(nxt_cw,),
                device_id_type=pl.DeviceIdType.MESH,
            )
            rdma_cw.start()

            c_ccw = lax.rem(r + s + 1, N_DEV)
            acc2 = partial_chunk(c_ccw, nh)
            if s > 0:
                acc2 = acc2 + recv_ccw[s - 1].astype(jnp.float32)
            send_ccw[...] = acc2.astype(jnp.bfloat16)
            rdma_ccw = pltpu.make_async_remote_copy(
                src_ref=send_ccw,
                dst_ref=recv_ccw.at[s],
                send_sem=send_sems.at[s, 1],
                recv_sem=recv_sems.at[s, 1],
                device_id=(nxt_ccw,),
                device_id_type=pl.DeviceIdType.MESH,
            )
            rdma_ccw.start()

            rdma_cw.wait()
            rdma_ccw.wait()

        fin_cw = partial_chunk(r, 0) + recv_cw[N_DEV - 2].astype(jnp.float32)
        fin_ccw = partial_chunk(r, nh) + recv_ccw[N_DEV - 2].astype(jnp.float32)
        out_ref[:, :nh] = fin_cw * jax.nn.sigmoid(fin_cw)
        out_ref[:, nh:] = fin_ccw * jax.nn.sigmoid(fin_ccw)

    return pl.pallas_call(
        body,
        out_shape=jax.ShapeDtypeStruct((m_per, n), jnp.float32),
        in_specs=[
            pl.BlockSpec(memory_space=pltpu.VMEM),
            pl.BlockSpec(memory_space=pltpu.VMEM),
        ],
        out_specs=pl.BlockSpec(memory_space=pltpu.VMEM),
        scratch_shapes=[
            pltpu.VMEM((m_per, nh), jnp.bfloat16),
            pltpu.VMEM((m_per, nh), jnp.bfloat16),
            pltpu.VMEM((N_DEV - 1, m_per, nh), jnp.bfloat16),
            pltpu.VMEM((N_DEV - 1, m_per, nh), jnp.bfloat16),
            pltpu.SemaphoreType.DMA((N_DEV - 1, 2)),
            pltpu.SemaphoreType.DMA((N_DEV - 1, 2)),
        ],
        compiler_params=pltpu.CompilerParams(collective_id=0),
    )(x, w_mat)


# device time: 53394 ns/iter; 2.3192x vs baseline; 1.5463x over previous
import jax
import jax.numpy as jnp
from jax import lax
from jax.experimental import pallas as pl
from jax.experimental.pallas import tpu as pltpu

N_DEV = 16
NSUB = 4

PERM = [0, 1, 5, 9, 13, 14, 10, 6, 2, 3, 7, 11, 15, 12, 8, 4]
INV = [PERM.index(l) for l in range(N_DEV)]
NXT_CW = [PERM[(INV[l] + 1) % N_DEV] for l in range(N_DEV)]
NXT_CCW = [PERM[(INV[l] - 1) % N_DEV] for l in range(N_DEV)]


def _lut(table, idx):
    acc = jnp.int32(0)
    for j, v in enumerate(table):
        acc = acc + jnp.where(idx == j, jnp.int32(v), jnp.int32(0))
    return acc


def kernel(x, w_mat):
    m, k_per = x.shape
    _, n = w_mat.shape
    m_per = m // N_DEV
    nh = n // 2
    nb = nh // NSUB

    def body(x_ref, w_ref, out_ref,
             send_cw, send_ccw, recv_cw, recv_ccw,
             send_sems, recv_sems):
        my = lax.axis_index("i")
        r = _lut(INV, my)
        nxt_cw = _lut(NXT_CW, my)
        nxt_ccw = _lut(NXT_CCW, my)

        barrier = pltpu.get_barrier_semaphore()
        for nbr in (nxt_cw, nxt_ccw):
            pl.semaphore_signal(
                barrier, inc=1,
                device_id=(nbr,), device_id_type=pl.DeviceIdType.MESH,
            )
        pl.semaphore_wait(barrier, 2)

        w = w_ref[...].astype(jnp.bfloat16)

        def partial_chunk(ring_chunk, col0):
            row = _lut(PERM, ring_chunk)
            xc = x_ref[pl.ds(row * m_per, m_per), :].astype(jnp.bfloat16)
            return jnp.dot(xc, w[:, col0:col0 + nh],
                           preferred_element_type=jnp.float32)

        def mk(dirn, s, b, target):
            sb, rb = (send_cw, recv_cw) if dirn == 0 else (send_ccw, recv_ccw)
            return pltpu.make_async_remote_copy(
                src_ref=sb.at[s, b],
                dst_ref=rb.at[s, b],
                send_sem=send_sems.at[s, b, dirn],
                recv_sem=recv_sems.at[s, b, dirn],
                device_id=(target,),
                device_id_type=pl.DeviceIdType.MESH,
            )

        for s in range(N_DEV - 1):
            c_cw = lax.rem(r + (2 * N_DEV - 1 - s), N_DEV)
            c_ccw = lax.rem(r + s + 1, N_DEV)
            part_cw = partial_chunk(c_cw, 0)
            part_ccw = partial_chunk(c_ccw, nh)

            for b in range(NSUB):
                cs = b * nb
                if s == 0:
                    val_cw = part_cw[:, cs:cs + nb]
                    val_ccw = part_ccw[:, cs:cs + nb]
                else:
                    mk(0, s - 1, b, nxt_cw).wait_recv()
                    val_cw = part_cw[:, cs:cs + nb] + \
                        recv_cw[s - 1, b].astype(jnp.float32)
                    mk(1, s - 1, b, nxt_ccw).wait_recv()
                    val_ccw = part_ccw[:, cs:cs + nb] + \
                        recv_ccw[s - 1, b].astype(jnp.float32)
                send_cw[s, b] = val_cw.astype(jnp.bfloat16)
                mk(0, s, b, nxt_cw).start()
                send_ccw[s, b] = val_ccw.astype(jnp.bfloat16)
                mk(1, s, b, nxt_ccw).start()

        part_cw = partial_chunk(r, 0)
        part_ccw = partial_chunk(r, nh)
        for b in range(NSUB):
            cs = b * nb
            mk(0, N_DEV - 2, b, nxt_cw).wait_recv()
            fin = part_cw[:, cs:cs + nb] + \
                recv_cw[N_DEV - 2, b].astype(jnp.float32)
            out_ref[:, cs:cs + nb] = fin * jax.nn.sigmoid(fin)
            mk(1, N_DEV - 2, b, nxt_ccw).wait_recv()
            fin2 = part_ccw[:, cs:cs + nb] + \
                recv_ccw[N_DEV - 2, b].astype(jnp.float32)
            out_ref[:, nh + cs:nh + cs + nb] = fin2 * jax.nn.sigmoid(fin2)

        for s in range(N_DEV - 1):
            for b in range(NSUB):
                mk(0, s, b, nxt_cw).wait_send()
                mk(1, s, b, nxt_ccw).wait_send()

    return pl.pallas_call(
        body,
        out_shape=jax.ShapeDtypeStruct((m_per, n), jnp.float32),
        in_specs=[
            pl.BlockSpec(memory_space=pltpu.VMEM),
            pl.BlockSpec(memory_space=pltpu.VMEM),
        ],
        out_specs=pl.BlockSpec(memory_space=pltpu.VMEM),
        scratch_shapes=[
            pltpu.VMEM((N_DEV - 1, NSUB, m_per, nb), jnp.bfloat16),
            pltpu.VMEM((N_DEV - 1, NSUB, m_per, nb), jnp.bfloat16),
            pltpu.VMEM((N_DEV - 1, NSUB, m_per, nb), jnp.bfloat16),
            pltpu.VMEM((N_DEV - 1, NSUB, m_per, nb), jnp.bfloat16),
            pltpu.SemaphoreType.DMA((N_DEV - 1, NSUB, 2)),
            pltpu.SemaphoreType.DMA((N_DEV - 1, NSUB, 2)),
        ],
        compiler_params=pltpu.CompilerParams(collective_id=0),
    )(x, w_mat)
